# baseline (speedup 1.0000x reference)
import math

import jax
import jax.numpy as jnp
from jax.experimental import pallas as pl

_RATIO = 0.5
_DEPTH = 4


def _blk(d):
    if d >= 4096:
        return 1024
    if d >= 2048:
        return 512
    if d >= 1024:
        return 256
    return 128


def _blk_k(d):
    return min(512, _blk(d))


def _ceil_to(d, b):
    return (d + b - 1) // b * b


def _padded(d):
    return _ceil_to(d, _blk(d))


def _loadmix(a_ref, b_ref):
    a, b = a_ref[...], b_ref[...]
    if a.dtype != b.dtype:
        a = a.astype(jnp.float32)
        b = b.astype(jnp.float32)
    return a, b


def _mm_kernel(a_ref, b_ref, o_ref):
    @pl.when(pl.program_id(2) == 0)
    def _init():
        o_ref[...] = jnp.zeros_like(o_ref)

    a, b = _loadmix(a_ref, b_ref)
    o_ref[...] += jnp.dot(a, b, preferred_element_type=jnp.float32)


def _mmT_kernel(a_ref, b_ref, o_ref):
    @pl.when(pl.program_id(2) == 0)
    def _init():
        o_ref[...] = jnp.zeros_like(o_ref)

    a, b = _loadmix(a_ref, b_ref)
    o_ref[...] += jax.lax.dot_general(
        a, b, (((0,), (0,)), ((), ())),
        preferred_element_type=jnp.float32)


def _pad2(arr, rp, cp):
    r, c = arr.shape
    if rp == r and cp == c:
        return arr
    return jnp.pad(arr, ((0, rp - r), (0, cp - c)))


def _mm(a, b, trans_a=False):
    if trans_a:
        K, M = a.shape
    else:
        M, K = a.shape
    K2, N = b.shape
    assert K == K2, (a.shape, b.shape, trans_a)
    bm, bn, bk = _blk(M), _blk(N), _blk_k(K)
    Mp, Np, Kp = _ceil_to(M, bm), _ceil_to(N, bn), _ceil_to(K, bk)
    if trans_a:
        a_p = _pad2(a, Kp, Mp)
        a_spec = pl.BlockSpec((bk, bm), lambda i, j, k: (k, i))
        kern = _mmT_kernel
    else:
        a_p = _pad2(a, Mp, Kp)
        a_spec = pl.BlockSpec((bm, bk), lambda i, j, k: (i, k))
        kern = _mm_kernel
    b_p = _pad2(b, Kp, Np)
    out = pl.pallas_call(
        kern,
        grid=(Mp // bm, Np // bn, Kp // bk),
        in_specs=[a_spec, pl.BlockSpec((bk, bn), lambda i, j, k: (k, j))],
        out_specs=pl.BlockSpec((bm, bn), lambda i, j, k: (i, j)),
        out_shape=jax.ShapeDtypeStruct((Mp, Np), jnp.float32),
    )(a_p, b_p)
    if Mp == M and Np == N:
        return out
    return out[:M, :N]


def kernel(x, edge_index, batch, clinical, W_down, b_down, p_pool, W_up, b_up):
    n0 = x.shape[0]
    n0p = _padded(n0)
    r, c = edge_index[0], edge_index[1]

    A0p = jnp.zeros((n0p, n0p), jnp.bfloat16).at[r, c].add(1.0)
    cnt = jnp.zeros((n0,), jnp.float32).at[c].add(1.0)
    dvec = jnp.zeros((n0,), jnp.float32).at[c].add(
        jnp.where(r == c, 1.0, 0.0))
    fill0 = jnp.where(dvec == 0.0, 2.0, 0.0)
    deg0 = cnt + fill0
    dinv0 = jnp.where(deg0 > 0, jax.lax.rsqrt(deg0), 0.0)
    u0 = jnp.pad(1.0 - dvec, (0, n0p - n0))

    def gcn0(xin, W, b):
        z = dinv0[:, None] * _mm(xin, W)
        zp = jnp.pad(z, ((0, n0p - n0), (0, 0)))
        agg = _mm(A0p, zp, trans_a=True)[:n0] + fill0[:, None] * z
        return dinv0[:, None] * agg + b[None, :]

    def gcn_dense(Ap, dinv, xin, W, b):
        k, kp = xin.shape[0], Ap.shape[0]
        z = dinv[:, None] * _mm(xin, W)
        zp = jnp.pad(z, ((0, kp - k), (0, 0)))
        agg = _mm(Ap, zp, trans_a=True)[:k] + 2.0 * z
        return dinv[:, None] * agg + b[None, :]

    xcur = jax.nn.relu(gcn0(x, W_down[0], b_down[0]))
    xs = [xcur]
    As = [None]
    dinvs = [None]
    perms = []
    A_prev = A0p
    u_prev = u0
    n_prev_p = n0p
    for i in range(1, _DEPTH + 1):
        n = xcur.shape[0]
        k = int(math.ceil(_RATIO * n))
        kp = _padded(k)
        p = p_pool[i - 1]
        score = jnp.tanh((xcur @ p) / jnp.linalg.norm(p))
        vals, perm = jax.lax.top_k(score, k)
        perm_p = jnp.concatenate(
            [perm, jnp.full((kp - k,), n_prev_p - 1, perm.dtype)])
        R = A_prev[perm_p, :]
        C = A_prev[:, perm_p]
        Arr = R[:, perm_p]
        P0 = _mm(R, C)
        if u_prev is None:
            corr = 2.0 * Arr
        else:
            urp = u_prev[perm_p]
            corr = urp[:, None] * Arr + urp[None, :] * Arr
        ii = jax.lax.broadcasted_iota(jnp.int32, (kp, kp), 0)
        jj = jax.lax.broadcasted_iota(jnp.int32, (kp, kp), 1)
        out_dt = jnp.bfloat16 if i <= 2 else jnp.float32
        Pp = jnp.where(ii != jj, P0 + corr, 0.0).astype(out_dt)
        x2 = xcur[perm] * vals[:, None]
        deg = Pp.sum(axis=0, dtype=jnp.float32)[:k] + 2.0
        dinv = jax.lax.rsqrt(deg)
        xcur = jax.nn.relu(gcn_dense(Pp, dinv, x2, W_down[i], b_down[i]))
        if i < _DEPTH:
            xs.append(xcur)
            As.append(Pp)
            dinvs.append(dinv)
        perms.append(perm)
        A_prev, u_prev, n_prev_p = Pp, None, kp

    for i in range(_DEPTH):
        j = _DEPTH - 1 - i
        res, perm = xs[j], perms[j]
        up = jnp.zeros_like(res).at[perm].set(xcur)
        xcur = res + up
        if j == 0:
            xcur = gcn0(xcur, W_up[i], b_up[i])
        else:
            xcur = gcn_dense(As[j], dinvs[j], xcur, W_up[i], b_up[i])
        if i < _DEPTH - 1:
            xcur = jax.nn.relu(xcur)

    return jnp.mean(xcur, axis=0, keepdims=True)


# speedup vs baseline: 1.1477x; 1.1477x over previous
import math

import jax
import jax.numpy as jnp
from jax.experimental import pallas as pl

_RATIO = 0.5
_DEPTH = 4


def _blk(d):
    if d >= 4096:
        return 1024
    if d >= 2048:
        return 512
    if d >= 1024:
        return 256
    return 128


def _blk_k(d):
    return min(512, _blk(d))


def _ceil_to(d, b):
    return (d + b - 1) // b * b


def _padded(d):
    return _ceil_to(d, _blk(d))


def _loadmix(a_ref, b_ref):
    a, b = a_ref[...], b_ref[...]
    if a.dtype != b.dtype:
        a = a.astype(jnp.float32)
        b = b.astype(jnp.float32)
    return a, b


def _mm_kernel(a_ref, b_ref, o_ref):
    @pl.when(pl.program_id(2) == 0)
    def _init():
        o_ref[...] = jnp.zeros_like(o_ref)

    a, b = _loadmix(a_ref, b_ref)
    o_ref[...] += jnp.dot(a, b, preferred_element_type=jnp.float32)


def _mmT_kernel(a_ref, b_ref, o_ref):
    @pl.when(pl.program_id(2) == 0)
    def _init():
        o_ref[...] = jnp.zeros_like(o_ref)

    a, b = _loadmix(a_ref, b_ref)
    o_ref[...] += jax.lax.dot_general(
        a, b, (((0,), (0,)), ((), ())),
        preferred_element_type=jnp.float32)


def _pad2(arr, rp, cp):
    r, c = arr.shape
    if rp == r and cp == c:
        return arr
    return jnp.pad(arr, ((0, rp - r), (0, cp - c)))


def _mm(a, b, trans_a=False):
    if trans_a:
        K, M = a.shape
    else:
        M, K = a.shape
    K2, N = b.shape
    assert K == K2, (a.shape, b.shape, trans_a)
    bm, bn, bk = _blk(M), _blk(N), _blk_k(K)
    Mp, Np, Kp = _ceil_to(M, bm), _ceil_to(N, bn), _ceil_to(K, bk)
    if trans_a:
        a_p = _pad2(a, Kp, Mp)
        a_spec = pl.BlockSpec((bk, bm), lambda i, j, k: (k, i))
        kern = _mmT_kernel
    else:
        a_p = _pad2(a, Mp, Kp)
        a_spec = pl.BlockSpec((bm, bk), lambda i, j, k: (i, k))
        kern = _mm_kernel
    b_p = _pad2(b, Kp, Np)
    out = pl.pallas_call(
        kern,
        grid=(Mp // bm, Np // bn, Kp // bk),
        in_specs=[a_spec, pl.BlockSpec((bk, bn), lambda i, j, k: (k, j))],
        out_specs=pl.BlockSpec((bm, bn), lambda i, j, k: (i, j)),
        out_shape=jax.ShapeDtypeStruct((Mp, Np), jnp.float32),
    )(a_p, b_p)
    if Mp == M and Np == N:
        return out
    return out[:M, :N]


def kernel(x, edge_index, batch, clinical, W_down, b_down, p_pool, W_up, b_up):
    n0 = x.shape[0]
    n0p = _padded(n0)
    r, c = edge_index[0], edge_index[1]

    A0p = jnp.zeros((n0p, n0p), jnp.float32).at[r, c].add(1.0)
    cnt = jnp.zeros((n0,), jnp.float32).at[c].add(1.0)
    dvec = jnp.zeros((n0,), jnp.float32).at[c].add(
        jnp.where(r == c, 1.0, 0.0))
    fill0 = jnp.where(dvec == 0.0, 2.0, 0.0)
    deg0 = cnt + fill0
    dinv0 = jnp.where(deg0 > 0, jax.lax.rsqrt(deg0), 0.0)
    u0 = jnp.pad(1.0 - dvec, (0, n0p - n0))

    def gcn0(xin, W, b):
        z = dinv0[:, None] * _mm(xin, W)
        zp = jnp.pad(z, ((0, n0p - n0), (0, 0)))
        agg = _mm(A0p, zp, trans_a=True)[:n0] + fill0[:, None] * z
        return dinv0[:, None] * agg + b[None, :]

    def gcn_dense(Ap, dinv, xin, W, b):
        k, kp = xin.shape[0], Ap.shape[0]
        z = dinv[:, None] * _mm(xin, W)
        zp = jnp.pad(z, ((0, kp - k), (0, 0)))
        agg = _mm(Ap, zp, trans_a=True)[:k] + 2.0 * z
        return dinv[:, None] * agg + b[None, :]

    xcur = jax.nn.relu(gcn0(x, W_down[0], b_down[0]))
    xs = [xcur]
    As = [None]
    dinvs = [None]
    perms = []
    A_prev = A0p
    u_prev = u0
    n_prev_p = n0p
    for i in range(1, _DEPTH + 1):
        n = xcur.shape[0]
        k = int(math.ceil(_RATIO * n))
        kp = _padded(k)
        p = p_pool[i - 1]
        score = jnp.tanh((xcur @ p) / jnp.linalg.norm(p))
        vals, perm = jax.lax.top_k(score, k)
        perm_p = jnp.concatenate(
            [perm, jnp.full((kp - k,), n_prev_p - 1, perm.dtype)])
        A_g = A_prev.astype(jnp.bfloat16) if i <= 2 else A_prev
        R = A_g[perm_p, :]
        C = A_g[:, perm_p]
        Arr = R[:, perm_p]
        P0 = _mm(R, C)
        if u_prev is None:
            corr = 2.0 * Arr
        else:
            urp = u_prev[perm_p]
            corr = urp[:, None] * Arr + urp[None, :] * Arr
        ii = jax.lax.broadcasted_iota(jnp.int32, (kp, kp), 0)
        jj = jax.lax.broadcasted_iota(jnp.int32, (kp, kp), 1)
        Pp = jnp.where(ii != jj, P0 + corr, 0.0)
        x2 = xcur[perm] * vals[:, None]
        deg = Pp.sum(axis=0)[:k] + 2.0
        dinv = jax.lax.rsqrt(deg)
        xcur = jax.nn.relu(gcn_dense(Pp, dinv, x2, W_down[i], b_down[i]))
        if i < _DEPTH:
            xs.append(xcur)
            As.append(Pp)
            dinvs.append(dinv)
        perms.append(perm)
        A_prev, u_prev, n_prev_p = Pp, None, kp

    for i in range(_DEPTH):
        j = _DEPTH - 1 - i
        res, perm = xs[j], perms[j]
        up = jnp.zeros_like(res).at[perm].set(xcur)
        xcur = res + up
        if j == 0:
            xcur = gcn0(xcur, W_up[i], b_up[i])
        else:
            xcur = gcn_dense(As[j], dinvs[j], xcur, W_up[i], b_up[i])
        if i < _DEPTH - 1:
            xcur = jax.nn.relu(xcur)

    return jnp.mean(xcur, axis=0, keepdims=True)
